# baseline (speedup 1.0000x reference)
import functools

import jax
import jax.numpy as jnp
from jax import lax
from jax.experimental import pallas as pl
from jax.experimental.pallas import tpu as pltpu
from jax.experimental.pallas import tpu_sc as plsc

_B = 16384
_D = 32
_NC = 2
_NS = 16
_NW = _NC * _NS
_BPW = _B // _NW


def _sc_body(uidx_hbm, iidx_hbm, utab_hbm, itab_hbm, out_hbm,
             uidx_v, iidx_v, urows_v, irows_v, s_u, s_i):
    c = lax.axis_index("c")
    s = lax.axis_index("s")
    wid = s * _NC + c
    base = wid * _BPW

    pltpu.sync_copy(uidx_hbm.at[pl.ds(base, _BPW)], uidx_v)
    pltpu.sync_copy(iidx_hbm.at[pl.ds(base, _BPW)], iidx_v)

    cu = pltpu.async_copy(utab_hbm.at[uidx_v], urows_v, s_u)
    ci = pltpu.async_copy(itab_hbm.at[iidx_v], irows_v, s_i)
    cu.wait()
    ci.wait()

    def row(r, carry):
        u0 = urows_v[r, pl.ds(0, 16)]
        i0 = irows_v[r, pl.ds(0, 16)]
        urows_v[r, pl.ds(0, 16)] = u0 * i0
        u1 = urows_v[r, pl.ds(16, 16)]
        i1 = irows_v[r, pl.ds(16, 16)]
        urows_v[r, pl.ds(16, 16)] = u1 * i1
        return carry

    lax.fori_loop(0, _BPW, row, 0)

    pltpu.sync_copy(urows_v, out_hbm.at[pl.ds(base, _BPW)])


_sc_gather_prod = pl.kernel(
    _sc_body,
    out_type=jax.ShapeDtypeStruct((_B, _D), jnp.float32),
    mesh=plsc.VectorSubcoreMesh(core_axis_name="c", subcore_axis_name="s"),
    scratch_types=[
        pltpu.VMEM((_BPW,), jnp.int32),
        pltpu.VMEM((_BPW,), jnp.int32),
        pltpu.VMEM((_BPW, _D), jnp.float32),
        pltpu.VMEM((_BPW, _D), jnp.float32),
        pltpu.SemaphoreType.DMA,
        pltpu.SemaphoreType.DMA,
    ],
    compiler_params=pltpu.CompilerParams(use_tc_tiling_on_sc=False),
    name="sc_gather_prod",
)


def _tc_body(p_ref, wt_ref, bt_ref, wo_ref, bo_ref, out_ref):
    h = jnp.dot(p_ref[...], wt_ref[...], preferred_element_type=jnp.float32)
    h = jnp.maximum(h + bt_ref[...], 0.0)
    out_ref[...] = jnp.dot(h, wo_ref[...], preferred_element_type=jnp.float32) + bo_ref[0, 0]


_tc_mlp = pl.pallas_call(
    _tc_body,
    out_shape=jax.ShapeDtypeStruct((_B, 1), jnp.float32),
    name="tc_mlp",
)


def kernel(user_idx, item_idx, user_table, item_table, W_t, b_t, W_o, b_o):
    prod = _sc_gather_prod(user_idx.astype(jnp.int32),
                           item_idx.astype(jnp.int32),
                           user_table, item_table)
    pred = _tc_mlp(prod, W_t,
                   b_t.reshape(1, _D),
                   W_o,
                   b_o.reshape(1, 1))
    return pred.reshape(_B)
